# baseline (speedup 1.0000x reference)
import math

import jax
import jax.numpy as jnp
from jax import lax
from jax.experimental import pallas as pl
from jax.experimental.pallas import tpu as pltpu
from jax.experimental.pallas import tpu_sc as plsc

D_MODEL = 64
SCALE = math.sqrt(D_MODEL)

_NC = 2
_NS = 16
_NW = _NC * _NS

_CHUNK = 512


def _emb_body(table_hbm, idx_hbm, out_hbm, idx_v, rows_v, gsem):
    wid = lax.axis_index("s") * _NC + lax.axis_index("c")
    bpw = idx_v.shape[0]
    base = wid * bpw
    pltpu.sync_copy(idx_hbm.at[pl.ds(base, bpw)], idx_v)

    nchunk = bpw // _CHUNK

    def chunk_body(c, carry):
        off = c * _CHUNK
        pltpu.async_copy(
            table_hbm.at[idx_v.at[pl.ds(off, _CHUNK)]], rows_v, gsem
        ).wait()

        def row_body(r, carry2):
            for j in range(D_MODEL // 16):
                sl = pl.ds(j * 16, 16)
                rows_v[r, sl] = rows_v[r, sl] * SCALE
            return carry2

        lax.fori_loop(0, _CHUNK, row_body, 0, unroll=4)

        pltpu.sync_copy(rows_v, out_hbm.at[pl.ds(base + off, _CHUNK)])
        return carry

    lax.fori_loop(0, nchunk, chunk_body, 0)


@jax.jit
def kernel(x, table):
    n_rows, n_cols = x.shape
    b = n_rows * n_cols
    bpw = b // _NW
    idx = x.reshape(b).astype(jnp.int32)

    mesh = plsc.VectorSubcoreMesh(core_axis_name="c", subcore_axis_name="s")
    fn = pl.kernel(
        _emb_body,
        out_type=jax.ShapeDtypeStruct((b, D_MODEL), jnp.float32),
        mesh=mesh,
        scratch_types=[
            pltpu.VMEM((bpw,), jnp.int32),
            pltpu.VMEM((_CHUNK, D_MODEL), jnp.float32),
            pltpu.SemaphoreType.DMA,
        ],
        compiler_params=pltpu.CompilerParams(use_tc_tiling_on_sc=False),
    )
    out = fn(table, idx)
    return out.reshape(n_rows, n_cols, D_MODEL)


# speedup vs baseline: 1.0677x; 1.0677x over previous
import math

import jax
import jax.numpy as jnp
from jax import lax
from jax.experimental import pallas as pl
from jax.experimental.pallas import tpu as pltpu
from jax.experimental.pallas import tpu_sc as plsc

D_MODEL = 64
SCALE = math.sqrt(D_MODEL)

_NC = 2
_NS = 16
_NW = _NC * _NS

_C = 320
_NBUF = 4


def _emb_body(table_hbm, idx_hbm, out_hbm, idx_v, *scratch):
    rows = scratch[0:_NBUF]
    gsem = scratch[_NBUF:2 * _NBUF]
    ssem = scratch[2 * _NBUF:3 * _NBUF]

    wid = lax.axis_index("s") * _NC + lax.axis_index("c")
    bpw = idx_v.shape[0]
    base = wid * bpw
    pltpu.sync_copy(idx_hbm.at[pl.ds(base, bpw)], idx_v)

    nchunk = bpw // _C
    ngroups = nchunk // _NBUF

    def start_gather(c, b):
        pltpu.async_copy(
            table_hbm.at[idx_v.at[pl.ds(c * _C, _C)]], rows[b], gsem[b]
        )

    def wait_gather(c, b):
        pltpu.make_async_copy(
            table_hbm.at[idx_v.at[pl.ds(c * _C, _C)]], rows[b], gsem[b]
        ).wait()

    def start_scatter(c, b):
        pltpu.async_copy(rows[b], out_hbm.at[pl.ds(base + c * _C, _C)], ssem[b])

    def wait_scatter(c, b):
        pltpu.make_async_copy(
            rows[b], out_hbm.at[pl.ds(base + c * _C, _C)], ssem[b]
        ).wait()

    def scale(b):
        buf = rows[b]

        def row_body(r, carry):
            for j in range(D_MODEL // 16):
                sl = pl.ds(j * 16, 16)
                buf[r, sl] = buf[r, sl] * SCALE
            return carry

        lax.fori_loop(0, _C, row_body, 0, unroll=8)

    for b in range(_NBUF):
        start_gather(b, b)

    def group(g, carry):
        c0 = g * _NBUF
        for b in range(_NBUF):
            wait_gather(c0 + b, b)
            scale(b)
            start_scatter(c0 + b, b)
        for b in range(_NBUF):
            wait_scatter(c0 + b, b)
            start_gather(c0 + b + _NBUF, b)
        return carry

    lax.fori_loop(0, ngroups - 1, group, 0)

    c0 = (ngroups - 1) * _NBUF
    for b in range(_NBUF):
        wait_gather(c0 + b, b)
        scale(b)
        start_scatter(c0 + b, b)
    for b in range(_NBUF):
        wait_scatter(c0 + b, b)


@jax.jit
def kernel(x, table):
    n_rows, n_cols = x.shape
    b = n_rows * n_cols
    bpw = b // _NW
    idx = x.reshape(b).astype(jnp.int32)

    mesh = plsc.VectorSubcoreMesh(core_axis_name="c", subcore_axis_name="s")
    scratch = (
        [pltpu.VMEM((bpw,), jnp.int32)]
        + [pltpu.VMEM((_C, D_MODEL), jnp.float32) for _ in range(_NBUF)]
        + [pltpu.SemaphoreType.DMA for _ in range(2 * _NBUF)]
    )
    fn = pl.kernel(
        _emb_body,
        out_type=jax.ShapeDtypeStruct((b, D_MODEL), jnp.float32),
        mesh=mesh,
        scratch_types=scratch,
        compiler_params=pltpu.CompilerParams(use_tc_tiling_on_sc=False),
    )
    out = fn(table, idx)
    return out.reshape(n_rows, n_cols, D_MODEL)
